# baseline (speedup 1.0000x reference)
import functools

import jax
import jax.numpy as jnp
from jax import lax
from jax.experimental import pallas as pl
from jax.experimental.pallas import tpu as pltpu
from jax.experimental.pallas import tpu_sc as plsc

B = 4096
F = 100
V = 10000
D = 100

NC = 2
NS = 16
NW = NC * NS
LANES = 16

ROWS = B * F
ROWS_PER_W = ROWS // NW
CHUNK = 512
NCHUNK = ROWS_PER_W // CHUNK
SUB = 128
NSUB = CHUNK // SUB


def _body(batch_hbm, table_hbm, out_hbm, idxraw_v, idx_v, rows_v, sem):
    wid = lax.axis_index("s") * NC + lax.axis_index("c")
    base = wid * ROWS_PER_W
    lane = lax.iota(jnp.int32, LANES)

    def chunk_step(c, _):
        r0 = base + c * CHUNK
        pltpu.sync_copy(batch_hbm.at[pl.ds(r0, CHUNK)], idxraw_v)

        def idx_step(i, _):
            off = i * LANES
            rid = r0 + off + lane
            f = lax.rem(rid, F)
            idx_v[pl.ds(off, LANES)] = idxraw_v[pl.ds(off, LANES)] + f * V
            return 0

        lax.fori_loop(0, CHUNK // LANES, idx_step, 0)

        copies = []
        for j in range(NSUB):
            copies.append(pltpu.async_copy(
                table_hbm.at[idx_v.at[pl.ds(j * SUB, SUB)]],
                rows_v.at[pl.ds(j * SUB, SUB)],
                sem,
            ))
        for cp in copies:
            cp.wait()
        pltpu.sync_copy(rows_v, out_hbm.at[pl.ds(r0, CHUNK)])
        return 0

    lax.fori_loop(0, NCHUNK, chunk_step, 0)


@functools.partial(
    pl.kernel,
    mesh=plsc.VectorSubcoreMesh(core_axis_name="c", subcore_axis_name="s"),
    out_type=jax.ShapeDtypeStruct((ROWS, D), jnp.float32),
    scratch_types=[
        pltpu.VMEM((CHUNK,), jnp.int32),
        pltpu.VMEM((CHUNK,), jnp.int32),
        pltpu.VMEM((CHUNK, D), jnp.float32),
        pltpu.SemaphoreType.DMA,
    ],
    compiler_params=pltpu.CompilerParams(use_tc_tiling_on_sc=False),
)
def _gather_kernel(batch_hbm, table_hbm, out_hbm, idxraw_v, idx_v, rows_v, sem):
    _body(batch_hbm, table_hbm, out_hbm, idxraw_v, idx_v, rows_v, sem)


def kernel(batch, tables):
    batch_flat = batch.reshape(ROWS)
    table_flat = tables.reshape(F * V, D)
    out = _gather_kernel(batch_flat, table_flat)
    return out.reshape(B, F, D)
